# Initial kernel scaffold; baseline (speedup 1.0000x reference)
import jax
import jax.numpy as jnp
from jax.experimental import pallas as pl


def kernel(x, edge_index, W1, b1, conv_w, W2, b2):
    raise NotImplementedError("write your pallas kernel here")


# speedup vs baseline: 6.9259x; 6.9259x over previous
import functools

import jax
import jax.numpy as jnp
import numpy as np
from jax import lax
from jax.experimental import pallas as pl
from jax.experimental.pallas import tpu as pltpu
from jax.experimental.pallas import tpu_sc as plsc

N = 10000
DIN = 128
HID = 128
DOUT = 128
NL = 8
ALPHA = 0.1
THETA = 0.5

NC = 2
NS = 16
NW = NC * NS
LANES = 16

NPAD = 10240
ROWS_PER_TILE = NPAD // NS
IDX_B = 128

_sc_mesh = plsc.VectorSubcoreMesh(
    core_axis_name="c", subcore_axis_name="s", num_cores=NC, num_subcores=NS)


def _chunks(e_total, nworkers):
  per_w = -(-e_total // nworkers)
  return -(-per_w // IDX_B)


def _pad_edges(idx, nworkers, ch, fill):
  e_pad = nworkers * ch * IDX_B
  p = jnp.concatenate([idx, jnp.full((e_pad - idx.shape[0],), fill, jnp.int32)])
  return p.reshape(nworkers, ch, IDX_B)


def _sc_deg_body(ch, dst_hbm, out_hbm, dst_v, ones_v, zeros_v, deg_sh):
  c = lax.axis_index("c")
  s = lax.axis_index("s")
  w = c * NS + s
  one = jnp.full((LANES,), 1.0, jnp.float32)
  zero = jnp.zeros((LANES,), jnp.float32)
  for k in range(IDX_B // LANES):
    ones_v[pl.ds(k * LANES, LANES)] = one
  def zbody(i, _):
    zeros_v[pl.ds(i * LANES, LANES)] = zero
    return 0
  lax.fori_loop(0, ROWS_PER_TILE // LANES, zbody, 0)
  base = s * ROWS_PER_TILE
  pltpu.sync_copy(zeros_v, deg_sh.at[pl.ds(base, ROWS_PER_TILE)])
  plsc.subcore_barrier()
  def body(j, _):
    pltpu.sync_copy(dst_hbm.at[w].at[j], dst_v)
    pltpu.sync_copy(ones_v, deg_sh.at[dst_v], add=True)
    return 0
  lax.fori_loop(0, ch, body, 0)
  plsc.subcore_barrier()
  pltpu.sync_copy(deg_sh.at[pl.ds(base, ROWS_PER_TILE)],
                  out_hbm.at[c].at[pl.ds(base, ROWS_PER_TILE)])


def _make_sc_deg(ch):
  return functools.partial(
      pl.kernel,
      out_type=jax.ShapeDtypeStruct((NC, NPAD), jnp.float32),
      mesh=_sc_mesh,
      scratch_types=[
          pltpu.VMEM((IDX_B,), jnp.int32),
          pltpu.VMEM((IDX_B,), jnp.float32),
          pltpu.VMEM((ROWS_PER_TILE,), jnp.float32),
          pltpu.VMEM_SHARED((NPAD,), jnp.float32),
      ],
      name="sc_deg_hist",
  )(functools.partial(_sc_deg_body, ch))


def _sc_agg_body(ch, src_hbm, dst_hbm, g_hbm, out_hbm,
                 src_v, dst_v, rows_v, zrow_v, agg_sh, gsem):
  c = lax.axis_index("c")
  s = lax.axis_index("s")
  w = c * NS + s
  zero = jnp.zeros((LANES,), jnp.float32)
  def zbody(i, _):
    for k in range(HID // LANES):
      zrow_v[i, pl.ds(k * LANES, LANES)] = zero
    return 0
  lax.fori_loop(0, IDX_B, zbody, 0)
  base = s * ROWS_PER_TILE
  for k in range(ROWS_PER_TILE // IDX_B):
    pltpu.sync_copy(zrow_v, agg_sh.at[pl.ds(base + k * IDX_B, IDX_B)])
  plsc.subcore_barrier()
  def body(j, _):
    pltpu.sync_copy(src_hbm.at[w].at[j], src_v)
    pltpu.sync_copy(dst_hbm.at[w].at[j], dst_v)
    pltpu.async_copy(g_hbm.at[src_v], rows_v, gsem).wait()
    pltpu.sync_copy(rows_v, agg_sh.at[dst_v], add=True)
    return 0
  lax.fori_loop(0, ch, body, 0)
  plsc.subcore_barrier()
  pltpu.sync_copy(agg_sh.at[pl.ds(base, ROWS_PER_TILE)],
                  out_hbm.at[c].at[pl.ds(base, ROWS_PER_TILE)])


def _make_sc_agg(ch):
  return functools.partial(
      pl.kernel,
      out_type=jax.ShapeDtypeStruct((NC, NPAD, HID), jnp.float32),
      mesh=_sc_mesh,
      scratch_types=[
          pltpu.VMEM((IDX_B,), jnp.int32),
          pltpu.VMEM((IDX_B,), jnp.int32),
          pltpu.VMEM((IDX_B, HID), jnp.float32),
          pltpu.VMEM((IDX_B, HID), jnp.float32),
          pltpu.VMEM_SHARED((NPAD, HID), jnp.float32),
          pltpu.SemaphoreType.DMA,
      ],
      name="sc_agg",
  )(functools.partial(_sc_agg_body, ch))


_BLK = 512
_GRID = NPAD // _BLK


def _tc_prep_body(x_ref, w1_ref, b1_ref, degp_ref, x0_ref, g_ref, dinv_ref):
  deg = degp_ref[0, :] + degp_ref[1, :] + 1.0
  dinv = lax.rsqrt(deg)
  dinv_b = jnp.broadcast_to(dinv[:, None], (_BLK, HID))
  h = jnp.maximum(
      jnp.dot(x_ref[...], w1_ref[...], preferred_element_type=jnp.float32)
      + b1_ref[...], 0.0)
  x0_ref[...] = h
  g_ref[...] = dinv_b * h
  dinv_ref[...] = dinv_b


def _tc_prep(x, w1, b1, degp):
  return pl.pallas_call(
      _tc_prep_body,
      grid=(_GRID,),
      in_specs=[
          pl.BlockSpec((_BLK, DIN), lambda i: (i, 0)),
          pl.BlockSpec((DIN, HID), lambda i: (0, 0)),
          pl.BlockSpec((1, HID), lambda i: (0, 0)),
          pl.BlockSpec((NC, _BLK), lambda i: (0, i)),
      ],
      out_specs=[
          pl.BlockSpec((_BLK, HID), lambda i: (i, 0)),
          pl.BlockSpec((_BLK, HID), lambda i: (i, 0)),
          pl.BlockSpec((_BLK, HID), lambda i: (i, 0)),
      ],
      out_shape=[
          jax.ShapeDtypeStruct((NPAD, HID), jnp.float32),
          jax.ShapeDtypeStruct((NPAD, HID), jnp.float32),
          jax.ShapeDtypeStruct((NPAD, HID), jnp.float32),
      ],
      name="tc_prep",
  )(x, w1, b1, degp)


def _tc_layer_body(beta, last, agg_ref, g_ref, x0_ref, dinv_ref, w_ref,
                   w2_ref, b2_ref, out_ref):
  ssum = agg_ref[0] + agg_ref[1] + g_ref[...]
  z = (1.0 - ALPHA) * (dinv_ref[...] * ssum) + ALPHA * x0_ref[...]
  t = (1.0 - beta) * z + beta * jnp.dot(
      z, w_ref[...], preferred_element_type=jnp.float32)
  h = jnp.maximum(t, 0.0)
  if last:
    out_ref[...] = jnp.dot(
        h, w2_ref[...], preferred_element_type=jnp.float32) + b2_ref[...]
  else:
    out_ref[...] = dinv_ref[...] * h


def _tc_layer(i, agg, g, x0, dinv, w, w2, b2):
  beta = float(np.log(THETA / (i + 1) + 1.0))
  last = (i == NL - 1)
  return pl.pallas_call(
      functools.partial(_tc_layer_body, beta, last),
      grid=(_GRID,),
      in_specs=[
          pl.BlockSpec((NC, _BLK, HID), lambda i: (0, i, 0)),
          pl.BlockSpec((_BLK, HID), lambda i: (i, 0)),
          pl.BlockSpec((_BLK, HID), lambda i: (i, 0)),
          pl.BlockSpec((_BLK, HID), lambda i: (i, 0)),
          pl.BlockSpec((HID, HID), lambda i: (0, 0)),
          pl.BlockSpec((HID, DOUT), lambda i: (0, 0)),
          pl.BlockSpec((1, DOUT), lambda i: (0, 0)),
      ],
      out_specs=pl.BlockSpec((_BLK, DOUT if last else HID), lambda i: (i, 0)),
      out_shape=jax.ShapeDtypeStruct((NPAD, DOUT if last else HID),
                                     jnp.float32),
      name=f"tc_layer_{i}",
  )(agg, g, x0, dinv, w, w2, b2)


def kernel(x, edge_index, W1, b1, conv_w, W2, b2):
  e_total = edge_index.shape[1]
  src = edge_index[0].astype(jnp.int32)
  dst = edge_index[1].astype(jnp.int32)

  ch = _chunks(e_total, NW)
  src_r = _pad_edges(src, NW, ch, 0)
  dst_r = _pad_edges(dst, NW, ch, N)

  x_pad = jnp.concatenate([x, jnp.zeros((NPAD - N, DIN), jnp.float32)])
  b1r = b1.reshape(1, HID)
  b2r = b2.reshape(1, DOUT)

  degp = _make_sc_deg(ch)(dst_r)
  x0, g, dinv = _tc_prep(x_pad, W1, b1r, degp)

  sc_agg = _make_sc_agg(ch)
  for i in range(NL):
    agg = sc_agg(src_r, dst_r, g)
    g = _tc_layer(i, agg, g, x0, dinv, conv_w[i], W2, b2r)
  return g[:N]
